# baseline (speedup 1.0000x reference)
import functools
import jax
import jax.numpy as jnp
from jax import lax
from jax.experimental import pallas as pl
from jax.experimental.pallas import tpu as pltpu
from jax.experimental.pallas import tpu_sc as plsc

N = 48758
E = 780128
D = 64
R = 4
B = 4096

NC = 2
NS = 16
NW = NC * NS
L = 16

NBLK = 48
NPAD = NBLK * 1024
DUMMY = N
K = 512
NB = 48
EPW = K * NB
EPAD = NW * EPW
RPT = NPAD // NS

_MESH = plsc.VectorSubcoreMesh(core_axis_name="c", subcore_axis_name="s")

_f32 = jnp.float32
_i32 = jnp.int32


def _fill_const(ref, n_rows, value):
    v = jnp.full((L,), value, _f32)

    def body(i, _):
        ref[i] = v
        return 0

    lax.fori_loop(0, n_rows, body, 0)


def _deg_body(col_hbm, deg_out, acc, colv, ones2d):
    cid = lax.axis_index("c")
    sid = lax.axis_index("s")
    wid = sid * NC + cid
    base = wid * EPW
    r0 = sid * RPT

    _fill_const(ones2d, K, 0.0)
    for j in range(RPT // K):
        pltpu.sync_copy(ones2d, acc.at[pl.ds(r0 + j * K, K)])
    _fill_const(ones2d, K, 1.0)
    plsc.subcore_barrier()

    def batch(b, _):
        pltpu.sync_copy(col_hbm.at[pl.ds(base + b * K, K)], colv)
        pltpu.sync_copy(ones2d, acc.at[colv], add=True)
        return 0

    lax.fori_loop(0, NB, batch, 0)
    plsc.subcore_barrier()
    pltpu.sync_copy(acc.at[pl.ds(r0, RPT)], deg_out.at[cid, pl.ds(r0, RPT)])


_SC_PARAMS = pltpu.CompilerParams(use_tc_tiling_on_sc=False)

_deg_kernel = functools.partial(
    pl.kernel,
    out_type=jax.ShapeDtypeStruct((NC, NPAD, L), _f32),
    mesh=_MESH,
    compiler_params=_SC_PARAMS,
    scratch_types=[
        pltpu.VMEM_SHARED((NPAD, L), _f32),
        pltpu.VMEM((K,), _i32),
        pltpu.VMEM((K, L), _f32),
    ],
)(_deg_body)


def _scan_body(row_hbm, col_hbm, rel_hbm, x01, x23, t0, t1, t2, t3,
               out, acc, rowv, colv, relv, gixv, ybuf, stg):
    xs = (x01, x23)
    ts = (t0, t1, t2, t3)
    cid = lax.axis_index("c")
    sid = lax.axis_index("s")
    wid = sid * NC + cid
    base = wid * EPW
    r0 = sid * RPT

    def zero_acc():
        def zf(i, _):
            z = jnp.zeros((L,), _f32)
            stg[i, 0:L] = z
            stg[i, L:2 * L] = z
            return 0

        lax.fori_loop(0, K, zf, 0)
        for j in range(RPT // K):
            pltpu.sync_copy(stg, acc.at[pl.ds(r0 + j * K, K)])
        plsc.subcore_barrier()

    def flush(j1):
        plsc.subcore_barrier()
        pltpu.sync_copy(acc.at[pl.ds(r0, RPT)], out.at[j1, cid, pl.ds(r0, RPT)])
        plsc.subcore_barrier()

    for p in range(2):
        zero_acc()

        def batch_a(b, _):
            pltpu.sync_copy(row_hbm.at[pl.ds(base + b * K, K)], rowv)
            pltpu.sync_copy(col_hbm.at[pl.ds(base + b * K, K)], colv)
            pltpu.sync_copy(xs[p].at[colv], stg)
            pltpu.sync_copy(stg, acc.at[rowv], add=True)
            return 0

        lax.fori_loop(0, NB, batch_a, 0)
        flush(p)

    for fc in range(4):
        zero_acc()

        def batch_b(b, _):
            pltpu.sync_copy(row_hbm.at[pl.ds(base + b * K, K)], rowv)
            pltpu.sync_copy(col_hbm.at[pl.ds(base + b * K, K)], colv)
            pltpu.sync_copy(rel_hbm.at[pl.ds(base + b * K, K)], relv)

            def gfill(i, _):
                s = pl.ds(i * L, L)
                gixv[s] = relv[s] * NPAD + colv[s]
                return 0

            lax.fori_loop(0, K // L, gfill, 0)
            pltpu.sync_copy(ts[fc].at[gixv], ybuf)

            def comp(k, _):
                y = ybuf[k]
                e = jnp.exp(y)
                stg[k, 0:L] = e
                stg[k, L:2 * L] = y * e
                return 0

            lax.fori_loop(0, K, comp, 0)
            pltpu.sync_copy(stg, acc.at[rowv], add=True)
            return 0

        lax.fori_loop(0, NB, batch_b, 0)
        flush(2 + fc)


_scan_kernel = functools.partial(
    pl.kernel,
    out_type=jax.ShapeDtypeStruct((6, NC, NPAD, 2 * L), _f32),
    mesh=_MESH,
    compiler_params=_SC_PARAMS,
    scratch_types=[
        pltpu.VMEM_SHARED((NPAD, 2 * L), _f32),
        pltpu.VMEM((K,), _i32),
        pltpu.VMEM((K,), _i32),
        pltpu.VMEM((K,), _i32),
        pltpu.VMEM((K,), _i32),
        pltpu.VMEM((K, L), _f32),
        pltpu.VMEM((K, 2 * L), _f32),
    ],
)(_scan_body)


def _take_body(h_hbm, idx_hbm, out_hbm, idxv, rows):
    cid = lax.axis_index("c")
    sid = lax.axis_index("s")
    wid = sid * NC + cid
    per = B // NW
    base = wid * per
    pltpu.sync_copy(idx_hbm.at[pl.ds(base, per)], idxv)
    pltpu.sync_copy(h_hbm.at[idxv], rows)
    pltpu.sync_copy(rows, out_hbm.at[pl.ds(base, per)])


_take_kernel = functools.partial(
    pl.kernel,
    out_type=jax.ShapeDtypeStruct((B, D), _f32),
    mesh=_MESH,
    compiler_params=_SC_PARAMS,
    scratch_types=[
        pltpu.VMEM((B // NW,), _i32),
        pltpu.VMEM((B // NW, D), _f32),
    ],
)(_take_body)


def _stats_body(x_ref, wp_ref, bp_ref, o_ref):
    i = pl.program_id(0)
    z = jnp.dot(x_ref[...], wp_ref[...].T, preferred_element_type=_f32) + bp_ref[...]
    s = jnp.sum(z, axis=0, keepdims=True)
    sq = jnp.sum(z * z, axis=0, keepdims=True)
    blk = jnp.concatenate([s, sq, jnp.zeros((6, D), _f32)], axis=0)

    @pl.when(i == 0)
    def _():
        o_ref[...] = blk

    @pl.when(i > 0)
    def _():
        o_ref[...] += blk


def _h_body(x_ref, wp_ref, bp_ref, g_ref, bt_ref, st_ref, o_ref):
    z = jnp.dot(x_ref[...], wp_ref[...].T, preferred_element_type=_f32) + bp_ref[...]
    npad_extra = float(NPAD - N)
    bp = bp_ref[...]
    ssum = st_ref[0:1, :] - npad_extra * bp
    ssq = st_ref[1:2, :] - npad_extra * bp * bp
    mu = ssum / float(N)
    var = ssq / float(N) - mu * mu
    hn = (z - mu) * lax.rsqrt(var + 1e-5) * g_ref[...] + bt_ref[...]
    o_ref[...] = jnp.maximum(hn, 0.0)


def _prep_body(h_ref, deg_ref, wi_ref, bi_ref, wr_ref,
               xo0, xo1, to0, to1, to2, to3):
    xos = (xo0, xo1)
    tos = (to0, to1, to2, to3)
    xl = jnp.dot(h_ref[...], wi_ref[...].T, preferred_element_type=_f32) + bi_ref[...]
    deg = deg_ref[0, :, 0:1] + deg_ref[1, :, 0:1]
    dis = jnp.where(deg > 0, lax.rsqrt(deg), 0.0)
    xsc = xl * dis
    for p in range(2):
        xos[p][...] = xsc[:, p * 2 * L:(p + 1) * 2 * L]
    for r in range(R):
        y = jnp.dot(xl, wr_ref[64 * r:64 * (r + 1), :], preferred_element_type=_f32)
        for fc in range(4):
            tos[fc][r] = y[:, fc * L:(fc + 1) * L]


def _finish_body(sc_ref, deg_ref, wo_ref, bo_ref, o_ref):
    deg = deg_ref[0, :, 0:1] + deg_ref[1, :, 0:1]
    dis = jnp.where(deg > 0, lax.rsqrt(deg), 0.0)
    acc = bo_ref[...]
    for f in range(4):
        g = sc_ref[f // 2, 0, :, (f % 2) * L:(f % 2 + 1) * L] \
            + sc_ref[f // 2, 1, :, (f % 2) * L:(f % 2 + 1) * L]
        den = sc_ref[2 + f, 0, :, 0:L] + sc_ref[2 + f, 1, :, 0:L]
        num = sc_ref[2 + f, 0, :, L:2 * L] + sc_ref[2 + f, 1, :, L:2 * L]
        msg = num / (den + 1e-16)
        t = g * dis + 0.1 * jnp.maximum(msg, 0.0)
        wslice = wo_ref[:, f * L:(f + 1) * L].T
        acc = acc + jnp.dot(t, wslice, preferred_element_type=_f32)
    o_ref[...] = acc


def _gelu_body(x_ref, o_ref):
    o_ref[...] = jax.nn.gelu(x_ref[...])


def _row_spec():
    return pl.BlockSpec((1024, D), lambda i: (i, 0))


def _w_spec(shape):
    return pl.BlockSpec(shape, lambda i: tuple(0 for _ in shape))


def _deg_spec():
    return pl.BlockSpec((NC, 1024, L), lambda i: (0, i, 0))


def _tc_stats(xpad, Wp, bp2):
    return pl.pallas_call(
        _stats_body,
        grid=(NBLK,),
        in_specs=[_row_spec(), _w_spec((D, D)), _w_spec((1, D))],
        out_specs=pl.BlockSpec((8, D), lambda i: (0, 0)),
        out_shape=jax.ShapeDtypeStruct((8, D), _f32),
    )(xpad, Wp, bp2)


def _tc_h(xpad, Wp, bp2, g2, bt2, stats):
    return pl.pallas_call(
        _h_body,
        grid=(NBLK,),
        in_specs=[_row_spec(), _w_spec((D, D)), _w_spec((1, D)),
                  _w_spec((1, D)), _w_spec((1, D)), _w_spec((8, D))],
        out_specs=_row_spec(),
        out_shape=jax.ShapeDtypeStruct((NPAD, D), _f32),
    )(xpad, Wp, bp2, g2, bt2, stats)


def _tc_prep(h, deg, Wi, bi2, Wr):
    xspec = pl.BlockSpec((1024, 2 * L), lambda i: (i, 0))
    tspec = pl.BlockSpec((R, 1024, L), lambda i: (0, i, 0))
    outs = pl.pallas_call(
        _prep_body,
        grid=(NBLK,),
        in_specs=[_row_spec(), _deg_spec(), _w_spec((D, D)), _w_spec((1, D)),
                  _w_spec((R * D, D))],
        out_specs=[xspec] * 2 + [tspec] * 4,
        out_shape=([jax.ShapeDtypeStruct((NPAD, 2 * L), _f32)] * 2
                   + [jax.ShapeDtypeStruct((R, NPAD, L), _f32)] * 4),
    )(h, deg, Wi, bi2, Wr)
    return outs[:2], [t.reshape(R * NPAD, L) for t in outs[2:]]


def _tc_finish(sc, deg, Wo, bo2):
    return pl.pallas_call(
        _finish_body,
        grid=(NBLK,),
        in_specs=[pl.BlockSpec((6, NC, 1024, 2 * L), lambda i: (0, 0, i, 0)),
                  _deg_spec(), _w_spec((D, D)), _w_spec((1, D))],
        out_specs=_row_spec(),
        out_shape=jax.ShapeDtypeStruct((NPAD, D), _f32),
    )(sc, deg, Wo, bo2)


def _tc_gelu(x):
    return pl.pallas_call(
        _gelu_body,
        grid=(B // 1024,),
        in_specs=[_row_spec()],
        out_specs=_row_spec(),
        out_shape=jax.ShapeDtypeStruct((B, D), _f32),
    )(x)


def kernel(x, edge_index, idx, edge_type, edge_weight, Wp, bp, bn_gamma, bn_beta,
           W_input0, b_input0, W_rel0, W_out0, b_out0,
           W_input1, b_input1, W_rel1, W_out1, b_out1):
    row = edge_index[0]
    col = edge_index[1]
    pad = EPAD - E
    rowp = jnp.concatenate([row, jnp.full((pad,), DUMMY, _i32)])
    colp = jnp.concatenate([col, jnp.full((pad,), DUMMY, _i32)])
    relp = jnp.concatenate([edge_type, jnp.zeros((pad,), _i32)])
    xpad = jnp.pad(x, ((0, NPAD - N), (0, 0)))

    bp2 = bp.reshape(1, D)
    g2 = bn_gamma.reshape(1, D)
    bt2 = bn_beta.reshape(1, D)

    deg = _deg_kernel(colp)

    stats = _tc_stats(xpad, Wp, bp2)
    h = _tc_h(xpad, Wp, bp2, g2, bt2, stats)

    for (Wi, bi, Wr, Wo, bo) in (
            (W_input0, b_input0, W_rel0, W_out0, b_out0),
            (W_input1, b_input1, W_rel1, W_out1, b_out1)):
        xs, ts = _tc_prep(h, deg, Wi, bi.reshape(1, D), Wr.reshape(R * D, D))
        sc = _scan_kernel(rowp, colp, relp, *xs, *ts)
        h = _tc_finish(sc, deg, Wo, bo.reshape(1, D))

    hb = _take_kernel(h, idx)
    return _tc_gelu(hb)


# speedup vs baseline: 1.3836x; 1.3836x over previous
import functools
import jax
import jax.numpy as jnp
from jax import lax
from jax.experimental import pallas as pl
from jax.experimental.pallas import tpu as pltpu
from jax.experimental.pallas import tpu_sc as plsc

N = 48758
E = 780128
D = 64
R = 4
B = 4096

NC = 2
NS = 16
NW = NC * NS
L = 16

NBLK = 48
NPAD = NBLK * 1024
DUMMY = N
K = 512
NB = 48
EPW = K * NB
EPAD = NW * EPW
RPT = NPAD // NS

_MESH = plsc.VectorSubcoreMesh(core_axis_name="c", subcore_axis_name="s")

_f32 = jnp.float32
_i32 = jnp.int32


def _fill_const(ref, n_rows, value):
    v = jnp.full((L,), value, _f32)

    def body(i, _):
        ref[i] = v
        return 0

    lax.fori_loop(0, n_rows, body, 0)


def _deg_body(col_hbm, deg_out, acc, colv, ones2d):
    cid = lax.axis_index("c")
    sid = lax.axis_index("s")
    wid = sid * NC + cid
    base = wid * EPW
    r0 = sid * RPT

    _fill_const(ones2d, K, 0.0)
    for j in range(RPT // K):
        pltpu.sync_copy(ones2d, acc.at[pl.ds(r0 + j * K, K)])
    _fill_const(ones2d, K, 1.0)
    plsc.subcore_barrier()

    def batch(b, _):
        pltpu.sync_copy(col_hbm.at[pl.ds(base + b * K, K)], colv)
        pltpu.sync_copy(ones2d, acc.at[colv], add=True)
        return 0

    lax.fori_loop(0, NB, batch, 0)
    plsc.subcore_barrier()
    pltpu.sync_copy(acc.at[pl.ds(r0, RPT)], deg_out.at[cid, pl.ds(r0, RPT)])


_SC_PARAMS = pltpu.CompilerParams(use_tc_tiling_on_sc=False)

_deg_kernel = functools.partial(
    pl.kernel,
    out_type=jax.ShapeDtypeStruct((NC, NPAD, L), _f32),
    mesh=_MESH,
    compiler_params=_SC_PARAMS,
    scratch_types=[
        pltpu.VMEM_SHARED((NPAD, L), _f32),
        pltpu.VMEM((K,), _i32),
        pltpu.VMEM((K, L), _f32),
    ],
)(_deg_body)


def _scan_body(row_hbm, col_hbm, rel_hbm, x0, x1, x2, x3, t0, t1, t2, t3,
               out, acc1, acc2, rowv, colv, relv, rowsc, gixv, ybuf, stg1, stg2,
               semi, semg0, semg1, semg2, semg3, semsc):
    xs = (x0, x1, x2, x3)
    ts = (t0, t1, t2, t3)
    semg = (semg0, semg1, semg2, semg3)
    cid = lax.axis_index("c")
    sid = lax.axis_index("s")
    wid = sid * NC + cid
    base = wid * EPW
    r0 = sid * RPT

    CH = 4
    KB = K // CH
    NCH = EPW // K

    def zero_accs():
        def zf(i, _):
            stg1[i] = jnp.zeros((L,), _f32)
            return 0

        lax.fori_loop(0, K, zf, 0)
        for j in range(RPT // K):
            pltpu.sync_copy(stg1, acc1.at[pl.ds(r0 + j * K, K)])
            pltpu.sync_copy(stg1, acc2.at[pl.ds(r0 + j * K, K)])
        plsc.subcore_barrier()

    def flush(j1, j2):
        plsc.subcore_barrier()
        pltpu.sync_copy(acc1.at[pl.ds(r0, RPT)], out.at[j1, cid, pl.ds(r0, RPT)])
        pltpu.sync_copy(acc2.at[pl.ds(r0, RPT)], out.at[j2, cid, pl.ds(r0, RPT)])
        plsc.subcore_barrier()

    def fill(j, with_gix):
        o = j * KB

        def ff(i, _):
            s = pl.ds(o + i * L, L)
            so = pl.ds(i * L, L)
            rowsc[j, so] = rowv[s]
            if with_gix:
                gixv[j, so] = relv[s] * NPAD + colv[s]
            return 0

        lax.fori_loop(0, KB // L, ff, 0)

    for p in range(2):
        zero_accs()

        def chunk_a(c, _):
            eb = base + c * K
            dl = []
            for j in range(CH):
                o = j * KB
                dl.append(pltpu.async_copy(
                    row_hbm.at[pl.ds(eb + o, KB)], rowv.at[pl.ds(o, KB)], semi))
                dl.append(pltpu.async_copy(
                    col_hbm.at[pl.ds(eb + o, KB)], colv.at[pl.ds(o, KB)], semi))
            for d in dl:
                d.wait()
            gl = []
            for j in range(CH):
                o = j * KB
                fill(j, False)
                gl.append(pltpu.async_copy(
                    xs[2 * p].at[colv.at[pl.ds(o, KB)]],
                    stg1.at[pl.ds(o, KB)], semg[j]))
                gl.append(pltpu.async_copy(
                    xs[2 * p + 1].at[colv.at[pl.ds(o, KB)]],
                    stg2.at[pl.ds(o, KB)], semg[j]))
            sl = []
            for j in range(CH):
                o = j * KB
                gl[2 * j].wait()
                gl[2 * j + 1].wait()
                d1 = pltpu.make_async_copy(
                    stg1.at[pl.ds(o, KB)], acc1.at[rowsc.at[j]], semsc)
                d1.start(add=True)
                d2 = pltpu.make_async_copy(
                    stg2.at[pl.ds(o, KB)], acc2.at[rowsc.at[j]], semsc)
                d2.start(add=True)
                sl.extend([d1, d2])
            for d in sl:
                d.wait()
            return 0

        lax.fori_loop(0, NCH, chunk_a, 0)
        flush(2 * p, 2 * p + 1)

    for fc in range(4):
        zero_accs()

        def chunk_b(c, _):
            eb = base + c * K
            dl = []
            for j in range(CH):
                o = j * KB
                dl.append(pltpu.async_copy(
                    row_hbm.at[pl.ds(eb + o, KB)], rowv.at[pl.ds(o, KB)], semi))
                dl.append(pltpu.async_copy(
                    col_hbm.at[pl.ds(eb + o, KB)], colv.at[pl.ds(o, KB)], semi))
                dl.append(pltpu.async_copy(
                    rel_hbm.at[pl.ds(eb + o, KB)], relv.at[pl.ds(o, KB)], semi))
            for d in dl:
                d.wait()
            gl = []
            for j in range(CH):
                o = j * KB
                fill(j, True)
                gl.append(pltpu.async_copy(
                    ts[fc].at[gixv.at[j]], ybuf.at[pl.ds(o, KB)], semg[j]))
            sl = []
            for j in range(CH):
                o = j * KB
                gl[j].wait()

                def cf(k, _):
                    y = ybuf[o + k]
                    e = jnp.exp(y)
                    stg1[o + k] = e
                    stg2[o + k] = y * e
                    return 0

                lax.fori_loop(0, KB, cf, 0)
                d1 = pltpu.make_async_copy(
                    stg1.at[pl.ds(o, KB)], acc1.at[rowsc.at[j]], semsc)
                d1.start(add=True)
                d2 = pltpu.make_async_copy(
                    stg2.at[pl.ds(o, KB)], acc2.at[rowsc.at[j]], semsc)
                d2.start(add=True)
                sl.extend([d1, d2])
            for d in sl:
                d.wait()
            return 0

        lax.fori_loop(0, NCH, chunk_b, 0)
        flush(4 + fc, 8 + fc)


_scan_kernel = functools.partial(
    pl.kernel,
    out_type=jax.ShapeDtypeStruct((12, NC, NPAD, L), _f32),
    mesh=_MESH,
    compiler_params=_SC_PARAMS,
    scratch_types=[
        pltpu.VMEM_SHARED((NPAD, L), _f32),
        pltpu.VMEM_SHARED((NPAD, L), _f32),
        pltpu.VMEM((K,), _i32),
        pltpu.VMEM((K,), _i32),
        pltpu.VMEM((K,), _i32),
        pltpu.VMEM((4, K // 4), _i32),
        pltpu.VMEM((4, K // 4), _i32),
        pltpu.VMEM((K, L), _f32),
        pltpu.VMEM((K, L), _f32),
        pltpu.VMEM((K, L), _f32),
        pltpu.SemaphoreType.DMA,
        pltpu.SemaphoreType.DMA,
        pltpu.SemaphoreType.DMA,
        pltpu.SemaphoreType.DMA,
        pltpu.SemaphoreType.DMA,
        pltpu.SemaphoreType.DMA,
    ],
)(_scan_body)


def _take_body(h_hbm, idx_hbm, out_hbm, idxv, rows):
    cid = lax.axis_index("c")
    sid = lax.axis_index("s")
    wid = sid * NC + cid
    per = B // NW
    base = wid * per
    pltpu.sync_copy(idx_hbm.at[pl.ds(base, per)], idxv)
    pltpu.sync_copy(h_hbm.at[idxv], rows)
    pltpu.sync_copy(rows, out_hbm.at[pl.ds(base, per)])


_take_kernel = functools.partial(
    pl.kernel,
    out_type=jax.ShapeDtypeStruct((B, D), _f32),
    mesh=_MESH,
    compiler_params=_SC_PARAMS,
    scratch_types=[
        pltpu.VMEM((B // NW,), _i32),
        pltpu.VMEM((B // NW, D), _f32),
    ],
)(_take_body)


def _stats_body(x_ref, wp_ref, bp_ref, o_ref):
    i = pl.program_id(0)
    z = jnp.dot(x_ref[...], wp_ref[...].T, preferred_element_type=_f32) + bp_ref[...]
    s = jnp.sum(z, axis=0, keepdims=True)
    sq = jnp.sum(z * z, axis=0, keepdims=True)
    blk = jnp.concatenate([s, sq, jnp.zeros((6, D), _f32)], axis=0)

    @pl.when(i == 0)
    def _():
        o_ref[...] = blk

    @pl.when(i > 0)
    def _():
        o_ref[...] += blk


def _h_body(x_ref, wp_ref, bp_ref, g_ref, bt_ref, st_ref, o_ref):
    z = jnp.dot(x_ref[...], wp_ref[...].T, preferred_element_type=_f32) + bp_ref[...]
    npad_extra = float(NPAD - N)
    bp = bp_ref[...]
    ssum = st_ref[0:1, :] - npad_extra * bp
    ssq = st_ref[1:2, :] - npad_extra * bp * bp
    mu = ssum / float(N)
    var = ssq / float(N) - mu * mu
    hn = (z - mu) * lax.rsqrt(var + 1e-5) * g_ref[...] + bt_ref[...]
    o_ref[...] = jnp.maximum(hn, 0.0)


def _prep_body(h_ref, deg_ref, wi_ref, bi_ref, wr_ref,
               xo0, xo1, xo2, xo3, to0, to1, to2, to3):
    xos = (xo0, xo1, xo2, xo3)
    tos = (to0, to1, to2, to3)
    xl = jnp.dot(h_ref[...], wi_ref[...].T, preferred_element_type=_f32) + bi_ref[...]
    deg = deg_ref[0, :, 0:1] + deg_ref[1, :, 0:1]
    dis = jnp.where(deg > 0, lax.rsqrt(deg), 0.0)
    xsc = xl * dis
    for fc in range(4):
        xos[fc][...] = xsc[:, fc * L:(fc + 1) * L]
    for r in range(R):
        y = jnp.dot(xl, wr_ref[64 * r:64 * (r + 1), :], preferred_element_type=_f32)
        for fc in range(4):
            tos[fc][r] = y[:, fc * L:(fc + 1) * L]


def _finish_body(sc_ref, deg_ref, wo_ref, bo_ref, o_ref):
    deg = deg_ref[0, :, 0:1] + deg_ref[1, :, 0:1]
    dis = jnp.where(deg > 0, lax.rsqrt(deg), 0.0)
    acc = bo_ref[...]
    for f in range(4):
        g = sc_ref[f, 0] + sc_ref[f, 1]
        den = sc_ref[4 + f, 0] + sc_ref[4 + f, 1]
        num = sc_ref[8 + f, 0] + sc_ref[8 + f, 1]
        msg = num / (den + 1e-16)
        t = g * dis + 0.1 * jnp.maximum(msg, 0.0)
        wslice = wo_ref[:, f * L:(f + 1) * L].T
        acc = acc + jnp.dot(t, wslice, preferred_element_type=_f32)
    o_ref[...] = acc


def _gelu_body(x_ref, o_ref):
    o_ref[...] = jax.nn.gelu(x_ref[...])


def _row_spec():
    return pl.BlockSpec((1024, D), lambda i: (i, 0))


def _w_spec(shape):
    return pl.BlockSpec(shape, lambda i: tuple(0 for _ in shape))


def _deg_spec():
    return pl.BlockSpec((NC, 1024, L), lambda i: (0, i, 0))


def _tc_stats(xpad, Wp, bp2):
    return pl.pallas_call(
        _stats_body,
        grid=(NBLK,),
        in_specs=[_row_spec(), _w_spec((D, D)), _w_spec((1, D))],
        out_specs=pl.BlockSpec((8, D), lambda i: (0, 0)),
        out_shape=jax.ShapeDtypeStruct((8, D), _f32),
    )(xpad, Wp, bp2)


def _tc_h(xpad, Wp, bp2, g2, bt2, stats):
    return pl.pallas_call(
        _h_body,
        grid=(NBLK,),
        in_specs=[_row_spec(), _w_spec((D, D)), _w_spec((1, D)),
                  _w_spec((1, D)), _w_spec((1, D)), _w_spec((8, D))],
        out_specs=_row_spec(),
        out_shape=jax.ShapeDtypeStruct((NPAD, D), _f32),
    )(xpad, Wp, bp2, g2, bt2, stats)


def _tc_prep(h, deg, Wi, bi2, Wr):
    xspec = pl.BlockSpec((1024, L), lambda i: (i, 0))
    tspec = pl.BlockSpec((R, 1024, L), lambda i: (0, i, 0))
    outs = pl.pallas_call(
        _prep_body,
        grid=(NBLK,),
        in_specs=[_row_spec(), _deg_spec(), _w_spec((D, D)), _w_spec((1, D)),
                  _w_spec((R * D, D))],
        out_specs=[xspec] * 4 + [tspec] * 4,
        out_shape=([jax.ShapeDtypeStruct((NPAD, L), _f32)] * 4
                   + [jax.ShapeDtypeStruct((R, NPAD, L), _f32)] * 4),
    )(h, deg, Wi, bi2, Wr)
    return outs[:4], [t.reshape(R * NPAD, L) for t in outs[4:]]


def _tc_finish(sc, deg, Wo, bo2):
    return pl.pallas_call(
        _finish_body,
        grid=(NBLK,),
        in_specs=[pl.BlockSpec((12, NC, 1024, L), lambda i: (0, 0, i, 0)),
                  _deg_spec(), _w_spec((D, D)), _w_spec((1, D))],
        out_specs=_row_spec(),
        out_shape=jax.ShapeDtypeStruct((NPAD, D), _f32),
    )(sc, deg, Wo, bo2)


def _tc_gelu(x):
    return pl.pallas_call(
        _gelu_body,
        grid=(B // 1024,),
        in_specs=[_row_spec()],
        out_specs=_row_spec(),
        out_shape=jax.ShapeDtypeStruct((B, D), _f32),
    )(x)


def kernel(x, edge_index, idx, edge_type, edge_weight, Wp, bp, bn_gamma, bn_beta,
           W_input0, b_input0, W_rel0, W_out0, b_out0,
           W_input1, b_input1, W_rel1, W_out1, b_out1):
    row = edge_index[0]
    col = edge_index[1]
    pad = EPAD - E
    rowp = jnp.concatenate([row, jnp.full((pad,), DUMMY, _i32)])
    colp = jnp.concatenate([col, jnp.full((pad,), DUMMY, _i32)])
    relp = jnp.concatenate([edge_type, jnp.zeros((pad,), _i32)])
    xpad = jnp.pad(x, ((0, NPAD - N), (0, 0)))

    bp2 = bp.reshape(1, D)
    g2 = bn_gamma.reshape(1, D)
    bt2 = bn_beta.reshape(1, D)

    deg = _deg_kernel(colp)

    stats = _tc_stats(xpad, Wp, bp2)
    h = _tc_h(xpad, Wp, bp2, g2, bt2, stats)

    for (Wi, bi, Wr, Wo, bo) in (
            (W_input0, b_input0, W_rel0, W_out0, b_out0),
            (W_input1, b_input1, W_rel1, W_out1, b_out1)):
        xs, ts = _tc_prep(h, deg, Wi, bi.reshape(1, D), Wr.reshape(R * D, D))
        sc = _scan_kernel(rowp, colp, relp, *xs, *ts)
        h = _tc_finish(sc, deg, Wo, bo.reshape(1, D))

    hb = _take_kernel(h, idx)
    return _tc_gelu(hb)
